# Initial kernel scaffold; baseline (speedup 1.0000x reference)
import jax
import jax.numpy as jnp
from jax.experimental import pallas as pl


def kernel(x, adj_inp, cheb_polynomials, L_tilde, pos_table, scale):
    raise NotImplementedError("write your pallas kernel here")


# speedup vs baseline: 1.2644x; 1.2644x over previous
import jax
import jax.numpy as jnp
from jax.experimental import pallas as pl


def _body(x_ref, pos_ref, scale_ref, o_ref):
    pos = pos_ref[...]
    sc = scale_ref[0, 0, :]
    pos_t = pos.T
    o_ref[...] = x_ref[...] + (sc[:, None] * pos_t)[None]


def kernel(x, adj_inp, cheb_polynomials, L_tilde, pos_table, scale):
    B, D, S, _ = x.shape
    x3 = x.reshape(B, D, S)
    d_blk, s_blk = 256, 512
    grid = (D // d_blk, S // s_blk, B)
    out = pl.pallas_call(
        _body,
        grid=grid,
        in_specs=[
            pl.BlockSpec((1, d_blk, s_blk), lambda i, j, b: (b, i, j)),
            pl.BlockSpec((s_blk, d_blk), lambda i, j, b: (j, i)),
            pl.BlockSpec((1, 1, d_blk), lambda i, j, b: (0, 0, i)),
        ],
        out_specs=pl.BlockSpec((1, d_blk, s_blk), lambda i, j, b: (b, i, j)),
        out_shape=jax.ShapeDtypeStruct((B, D, S), jnp.float32),
    )(x3, pos_table, scale)
    return out.reshape(B, D, S, 1)
